# Initial kernel scaffold; baseline (speedup 1.0000x reference)
import jax
import jax.numpy as jnp
from jax.experimental import pallas as pl


def kernel(x, edge_index, W, b):
    raise NotImplementedError("write your pallas kernel here")


# speedup vs baseline: 2.4472x; 2.4472x over previous
import functools

import jax
import jax.numpy as jnp
from jax import lax
from jax.experimental import pallas as pl
from jax.experimental.pallas import tpu as pltpu
from jax.experimental.pallas import tpu_sc as plsc

_C = 128
_K = 32
_N = 10000
_NW = 32
_NPW = 320
_NPAD = _NW * _NPW


def _tc_tables_body(x_ref, w_ref, b_ref, ga_ref, gb_ref):
    xt = x_ref[...].T
    w1 = w_ref[:, :_C]
    w2 = w_ref[:, _C:]
    dn = (((1,), (0,)), ((), ()))
    ga = lax.dot_general(xt, (w1 - w2).T, dn,
                         preferred_element_type=jnp.float32,
                         precision=lax.Precision.HIGHEST)
    gb = lax.dot_general(xt, w2.T, dn,
                         preferred_element_type=jnp.float32,
                         precision=lax.Precision.HIGHEST)
    ga_ref[...] = ga + b_ref[...][None, :]
    gb_ref[...] = gb


def _tc_tables(x2d, W, b):
    return pl.pallas_call(
        _tc_tables_body,
        out_shape=(
            jax.ShapeDtypeStruct((_N, _C), jnp.float32),
            jax.ShapeDtypeStruct((_N, _C), jnp.float32),
        ),
    )(x2d, W, b)


def _sc_body(ga, gb, idxa, idxb, out,
             idxa_v, idxb_v, arows, brows, out_v, sema, semb):
    wid = lax.axis_index("s") * 2 + lax.axis_index("c")
    base = wid * _NPW
    pltpu.sync_copy(idxa.at[pl.ds(base, _NPW)], idxa_v)
    pltpu.sync_copy(idxb.at[pl.ds(base, _NPW)], idxb_v)

    def node_body(n, carry):
        ca = pltpu.async_copy(ga.at[idxa_v.at[n]], arows, sema)
        cb = pltpu.async_copy(gb.at[idxb_v.at[n]], brows, semb)
        ca.wait()
        cb.wait()
        zero = jnp.zeros((16,), jnp.float32)
        for c in range(_C // 16):
            s = pl.ds(c * 16, 16)
            acc = arows[0, s] + brows[0, s]
            for k in range(1, _K):
                acc = jnp.maximum(acc, arows[k, s] + brows[k, s])
            out_v[n, s] = jnp.maximum(acc, zero)
        return carry

    lax.fori_loop(0, _NPW, node_body, 0)
    pltpu.sync_copy(out_v, out.at[pl.ds(base, _NPW)])


@functools.cache
def _sc_gather_max():
    return pl.kernel(
        _sc_body,
        out_type=jax.ShapeDtypeStruct((_NPAD, _C), jnp.float32),
        mesh=plsc.VectorSubcoreMesh(core_axis_name="c", subcore_axis_name="s"),
        scratch_types=[
            pltpu.VMEM((_NPW, _K), jnp.int32),
            pltpu.VMEM((_NPW, _K), jnp.int32),
            pltpu.VMEM((_K, _C), jnp.float32),
            pltpu.VMEM((_K, _C), jnp.float32),
            pltpu.VMEM((_NPW, _C), jnp.float32),
            pltpu.SemaphoreType.DMA,
            pltpu.SemaphoreType.DMA,
        ],
    )


def kernel(x, edge_index, W, b):
    B = x.shape[0]
    x2d = x.reshape(_C, _N)
    idx0 = edge_index[0].reshape(_N, _K).astype(jnp.int32)
    idx1 = edge_index[1].reshape(_N, _K).astype(jnp.int32)
    pad = ((0, _NPAD - _N), (0, 0))
    idxa = jnp.pad(idx1, pad)
    idxb = jnp.pad(idx0, pad)

    ga, gb = _tc_tables(x2d, W, b)
    out_rows = _sc_gather_max()(ga, gb, idxa, idxb)
    return out_rows[:_N].T.reshape(B, _C, _N, 1)


# speedup vs baseline: 3.2202x; 1.3159x over previous
import functools

import jax
import jax.numpy as jnp
from jax import lax
from jax.experimental import pallas as pl
from jax.experimental.pallas import tpu as pltpu
from jax.experimental.pallas import tpu_sc as plsc

_C = 128
_K = 32
_N = 10000
_NW = 32
_NPW = 320
_NPAD = _NW * _NPW


def _tc_tables_body(x_ref, w_ref, b_ref, ga_ref, gb_ref):
    xt = x_ref[...].T
    w1 = w_ref[:, :_C]
    w2 = w_ref[:, _C:]
    dn = (((1,), (0,)), ((), ()))
    ga = lax.dot_general(xt, (w1 - w2).T, dn,
                         preferred_element_type=jnp.float32,
                         precision=lax.Precision.HIGHEST)
    gb = lax.dot_general(xt, w2.T, dn,
                         preferred_element_type=jnp.float32,
                         precision=lax.Precision.HIGHEST)
    ga_ref[...] = ga + b_ref[...][None, :]
    gb_ref[...] = gb


def _tc_tables(x2d, W, b):
    return pl.pallas_call(
        _tc_tables_body,
        out_shape=(
            jax.ShapeDtypeStruct((_N, _C), jnp.float32),
            jax.ShapeDtypeStruct((_N, _C), jnp.float32),
        ),
    )(x2d, W, b)


_CH = 4
_NCH = _NPW // _CH
_RK = _CH * _K


def _sc_body(ga, gb, idxa, idxb, out,
             idxa_v, idxb_v, ar0, br0, ar1, br1, out_v,
             semA0, semB0, semA1, semB1):
    wid = lax.axis_index("s") * 2 + lax.axis_index("c")
    base = wid * _NPW
    cbase = wid * _NCH
    pltpu.sync_copy(idxa.at[pl.ds(cbase, _NCH)], idxa_v)
    pltpu.sync_copy(idxb.at[pl.ds(cbase, _NCH)], idxb_v)

    slots = ((ar0, br0, semA0, semB0), (ar1, br1, semA1, semB1))

    def issue(g, slot):
        arows, brows, sa, sb = slot
        pltpu.async_copy(ga.at[idxa_v.at[g]], arows, sa)
        pltpu.async_copy(gb.at[idxb_v.at[g]], brows, sb)

    def wait(slot):
        arows, brows, sa, sb = slot
        pltpu.make_async_copy(ga.at[pl.ds(0, _RK)], arows, sa).wait()
        pltpu.make_async_copy(gb.at[pl.ds(0, _RK)], brows, sb).wait()

    def compute(g, slot):
        arows, brows, _, _ = slot
        zero = jnp.zeros((16,), jnp.float32)
        for ni in range(_CH):
            n = g * _CH + ni
            for c in range(_C // 16):
                s = pl.ds(c * 16, 16)
                acc = arows[ni * _K, s] + brows[ni * _K, s]
                for k in range(1, _K):
                    acc = jnp.maximum(acc, arows[ni * _K + k, s] +
                                      brows[ni * _K + k, s])
                out_v[n, s] = jnp.maximum(acc, zero)

    issue(0, slots[0])
    issue(1, slots[1])

    def pair_body(t, carry):
        for p in range(2):
            g = 2 * t + p
            wait(slots[p])
            compute(g, slots[p])

            @pl.when(g + 2 < _NCH)
            def _():
                issue(g + 2, slots[p])
        return carry

    lax.fori_loop(0, _NCH // 2, pair_body, 0)
    pltpu.sync_copy(out_v, out.at[pl.ds(base, _NPW)])


@functools.cache
def _sc_gather_max():
    return pl.kernel(
        _sc_body,
        out_type=jax.ShapeDtypeStruct((_NPAD, _C), jnp.float32),
        mesh=plsc.VectorSubcoreMesh(core_axis_name="c", subcore_axis_name="s"),
        scratch_types=[
            pltpu.VMEM((_NCH, _CH * _K), jnp.int32),
            pltpu.VMEM((_NCH, _CH * _K), jnp.int32),
            pltpu.VMEM((_RK, _C), jnp.float32),
            pltpu.VMEM((_RK, _C), jnp.float32),
            pltpu.VMEM((_RK, _C), jnp.float32),
            pltpu.VMEM((_RK, _C), jnp.float32),
            pltpu.VMEM((_NPW, _C), jnp.float32),
            pltpu.SemaphoreType.DMA,
            pltpu.SemaphoreType.DMA,
            pltpu.SemaphoreType.DMA,
            pltpu.SemaphoreType.DMA,
        ],
    )


def kernel(x, edge_index, W, b):
    B = x.shape[0]
    x2d = x.reshape(_C, _N)
    idx0 = edge_index[0].reshape(_N, _K).astype(jnp.int32)
    idx1 = edge_index[1].reshape(_N, _K).astype(jnp.int32)
    pad = ((0, _NPAD - _N), (0, 0))
    idxa = jnp.pad(idx1, pad).reshape(_NPAD // _CH, _CH * _K)
    idxb = jnp.pad(idx0, pad).reshape(_NPAD // _CH, _CH * _K)

    ga, gb = _tc_tables(x2d, W, b)
    out_rows = _sc_gather_max()(ga, gb, idxa, idxb)
    return out_rows[:_N].T.reshape(B, _C, _N, 1)


# speedup vs baseline: 4.3052x; 1.3369x over previous
import functools

import jax
import jax.numpy as jnp
import numpy as np
from jax import lax
from jax.experimental import pallas as pl
from jax.experimental.pallas import tpu as pltpu
from jax.experimental.pallas import tpu_sc as plsc

_C = 128
_K = 32
_N = 10000
_NW = 32
_NPW = 320
_NPAD = _NW * _NPW
_CW = _C // 2

_CH = 4
_NCH = _NPW // _CH
_RK = _CH * _K

_PERM = np.empty(_C, dtype=np.int32)
_PERM[0::2] = np.arange(_CW)
_PERM[1::2] = _CW + np.arange(_CW)


def _tc_tables_body(x_ref, w_ref, b_ref, ga_ref, gb_ref):
    xt = x_ref[...].T
    w1 = w_ref[:, :_C]
    w2 = w_ref[:, _C:]
    dn = (((1,), (0,)), ((), ()))
    ga = lax.dot_general(xt, (w1 - w2).T, dn,
                         preferred_element_type=jnp.float32,
                         precision=lax.Precision.HIGHEST)
    gb = lax.dot_general(xt, w2.T, dn,
                         preferred_element_type=jnp.float32,
                         precision=lax.Precision.HIGHEST)
    ga_ref[...] = (ga + b_ref[...][None, :]).astype(jnp.bfloat16)
    gb_ref[...] = gb.astype(jnp.bfloat16)


def _tc_tables(x2d, Wp, bp):
    return pl.pallas_call(
        _tc_tables_body,
        out_shape=(
            jax.ShapeDtypeStruct((_N, _C), jnp.bfloat16),
            jax.ShapeDtypeStruct((_N, _C), jnp.bfloat16),
        ),
    )(x2d, Wp, bp)


def _sc_body(ga, gb, idxa, idxb, out,
             idxa_v, idxb_v, ar0, br0, ar1, br1, out_v,
             semA0, semB0, semA1, semB1):
    wid = lax.axis_index("s") * 2 + lax.axis_index("c")
    base = wid * _NPW
    cbase = wid * _NCH
    pltpu.sync_copy(idxa.at[pl.ds(cbase, _NCH)], idxa_v)
    pltpu.sync_copy(idxb.at[pl.ds(cbase, _NCH)], idxb_v)

    slots = ((ar0, br0, semA0, semB0), (ar1, br1, semA1, semB1))

    def issue(g, slot):
        arows, brows, sa, sb = slot
        pltpu.async_copy(ga.at[idxa_v.at[g]], arows, sa)
        pltpu.async_copy(gb.at[idxb_v.at[g]], brows, sb)

    def wait(slot):
        arows, brows, sa, sb = slot
        pltpu.make_async_copy(ga.at[pl.ds(0, _RK)], arows, sa).wait()
        pltpu.make_async_copy(gb.at[pl.ds(0, _RK)], brows, sb).wait()

    himask = jnp.full((16,), -65536, jnp.int32)

    def halves(w):
        lo = lax.bitcast_convert_type(lax.shift_left(w, 16), jnp.float32)
        hi = lax.bitcast_convert_type(lax.bitwise_and(w, himask), jnp.float32)
        return lo, hi

    def compute(g, slot):
        arows, brows, _, _ = slot
        zero = jnp.zeros((16,), jnp.float32)
        for ni in range(_CH):
            n = g * _CH + ni
            r = ni * _K
            for c in range(_CW // 16):
                s = pl.ds(c * 16, 16)
                alo, ahi = halves(arows[r, s])
                blo, bhi = halves(brows[r, s])
                acc_lo = alo + blo
                acc_hi = ahi + bhi
                for k in range(1, _K):
                    alo, ahi = halves(arows[r + k, s])
                    blo, bhi = halves(brows[r + k, s])
                    acc_lo = jnp.maximum(acc_lo, alo + blo)
                    acc_hi = jnp.maximum(acc_hi, ahi + bhi)
                out_v[n, pl.ds(c * 16, 16)] = jnp.maximum(acc_lo, zero)
                out_v[n, pl.ds(_CW + c * 16, 16)] = jnp.maximum(acc_hi, zero)

    issue(0, slots[0])
    issue(1, slots[1])

    def pair_body(t, carry):
        for p in range(2):
            g = 2 * t + p
            wait(slots[p])
            compute(g, slots[p])

            @pl.when(g + 2 < _NCH)
            def _():
                issue(g + 2, slots[p])
        return carry

    lax.fori_loop(0, _NCH // 2, pair_body, 0)
    pltpu.sync_copy(out_v, out.at[pl.ds(base, _NPW)])


@functools.cache
def _sc_gather_max():
    return pl.kernel(
        _sc_body,
        out_type=jax.ShapeDtypeStruct((_NPAD, _C), jnp.float32),
        mesh=plsc.VectorSubcoreMesh(core_axis_name="c", subcore_axis_name="s"),
        compiler_params=pltpu.CompilerParams(use_tc_tiling_on_sc=False),
        scratch_types=[
            pltpu.VMEM((_NCH, _CH * _K), jnp.int32),
            pltpu.VMEM((_NCH, _CH * _K), jnp.int32),
            pltpu.VMEM((_RK, _CW), jnp.int32),
            pltpu.VMEM((_RK, _CW), jnp.int32),
            pltpu.VMEM((_RK, _CW), jnp.int32),
            pltpu.VMEM((_RK, _CW), jnp.int32),
            pltpu.VMEM((_NPW, _C), jnp.float32),
            pltpu.SemaphoreType.DMA,
            pltpu.SemaphoreType.DMA,
            pltpu.SemaphoreType.DMA,
            pltpu.SemaphoreType.DMA,
        ],
    )


def kernel(x, edge_index, W, b):
    B = x.shape[0]
    x2d = x.reshape(_C, _N)
    idx0 = edge_index[0].reshape(_N, _K).astype(jnp.int32)
    idx1 = edge_index[1].reshape(_N, _K).astype(jnp.int32)
    pad = ((0, _NPAD - _N), (0, 0))
    idxa = jnp.pad(idx1, pad).reshape(_NPAD // _CH, _CH * _K)
    idxb = jnp.pad(idx0, pad).reshape(_NPAD // _CH, _CH * _K)

    perm = jnp.asarray(_PERM)
    ga_bf, gb_bf = _tc_tables(x2d, W[perm, :], b[perm])
    ga_w = lax.bitcast_convert_type(ga_bf.reshape(_N, _CW, 2), jnp.int32)
    gb_w = lax.bitcast_convert_type(gb_bf.reshape(_N, _CW, 2), jnp.int32)
    out_rows = _sc_gather_max()(ga_w, gb_w, idxa, idxb)
    return out_rows[:_N].T.reshape(B, _C, _N, 1)
